# baseline (speedup 1.0000x reference)
import functools

import numpy as np
import jax
import jax.numpy as jnp
from jax import lax
from jax.experimental import pallas as pl
from jax.experimental.pallas import tpu as pltpu
from jax.experimental.pallas import tpu_sc as plsc

N = 10000
E = 320000
D = 128
MSG = 256
HID = 384
NC = 2
NS = 16
K = 40
EPW = E // (NC * NS)
CH = EPW // K
D2 = D // 2

UNPACK_PERM = np.arange(MSG).reshape(MSG // 32, 16, 2).transpose(0, 2, 1).reshape(MSG)
NPAD = 10240
STRIPE = NPAD // NS
RB = 400


def _ab_body(x_ref, w1_ref, w2_ref, b_ref, a0_ref, a1_ref, b0_ref, b1_ref):
    xb = x_ref[...]
    dn = (((1,), (1,)), ((), ()))
    a = lax.dot_general(xb, w1_ref[...], dn, preferred_element_type=jnp.float32)
    b = lax.dot_general(xb, w2_ref[...], dn, preferred_element_type=jnp.float32)
    b = b + b_ref[...]
    a0_ref[...] = a[:, :D].astype(jnp.bfloat16)
    a1_ref[...] = a[:, D:].astype(jnp.bfloat16)
    b0_ref[...] = b[:, :D].astype(jnp.bfloat16)
    b1_ref[...] = b[:, D:].astype(jnp.bfloat16)


def _project_ab(x, w_src, w_dst, b_msg):
    half = pl.BlockSpec((RB, D), lambda i: (i, 0))
    return pl.pallas_call(
        _ab_body,
        grid=(N // RB,),
        in_specs=[
            pl.BlockSpec((RB, D), lambda i: (i, 0)),
            pl.BlockSpec((MSG, D), lambda i: (0, 0)),
            pl.BlockSpec((MSG, D), lambda i: (0, 0)),
            pl.BlockSpec((1, MSG), lambda i: (0, 0)),
        ],
        out_specs=[half, half, half, half],
        out_shape=[jax.ShapeDtypeStruct((N, D), jnp.bfloat16)] * 4,
    )(x, w_src, w_dst, b_msg)


def _sc_body(a0, a1, b0, b1, src2d, dst2d, part,
             acc, srcb, dstb, arow, brow, mrow, zbuf,
             sem_a0, sem_a1, sem_b0, sem_b1):
    sems_a = (sem_a0, sem_a1)
    sems_b = (sem_b0, sem_b1)
    c = lax.axis_index("c")
    s = lax.axis_index("s")
    wid = c * NS + s

    pltpu.sync_copy(src2d.at[wid], srcb)
    pltpu.sync_copy(dst2d.at[wid], dstb)

    zero = jnp.zeros((16,), jnp.float32)

    def zfill(i, _):
        for f in range(D // 16):
            zbuf[i, pl.ds(f * 16, 16)] = zero
        return 0
    lax.fori_loop(0, 64, zfill, 0)

    for fh in range(2):
        a_hbm = a0 if fh == 0 else a1
        b_hbm = b0 if fh == 0 else b1

        def zbody(i, _):
            pltpu.sync_copy(zbuf, acc.at[pl.ds(s * STRIPE + i * 64, 64)])
            return 0
        lax.fori_loop(0, STRIPE // 64, zbody, 0)
        plsc.subcore_barrier()

        def fire(j, buf):
            pltpu.async_copy(a_hbm.at[srcb.at[j]], arow.at[buf], sems_a[buf])
            pltpu.async_copy(b_hbm.at[dstb.at[j]], brow.at[buf], sems_b[buf])

        def drain(j, buf):
            pltpu.make_async_copy(a_hbm.at[srcb.at[j]], arow.at[buf],
                                  sems_a[buf]).wait()
            pltpu.make_async_copy(b_hbm.at[dstb.at[j]], brow.at[buf],
                                  sems_b[buf]).wait()

        fire(0, 0)

        def cbody(j2, _):
            for b in range(2):
                j = j2 * 2 + b
                nxt = j + 1

                @pl.when(nxt < CH)
                def _prefetch():
                    fire(nxt, 1 - b)

                drain(j, b)

                def ebody(e2, _):
                    for de in range(2):
                        e = e2 * 2 + de
                        for f in range(D2 // 16):
                            ra = arow[b, e, pl.ds(f * 16, 16)]
                            rb = brow[b, e, pl.ds(f * 16, 16)]
                            msk = jnp.int32(-65536)
                            bc = lambda t: lax.bitcast_convert_type(
                                t, jnp.float32)
                            ev = bc(ra << 16) + bc(rb << 16)
                            od = bc(ra & msk) + bc(rb & msk)
                            mrow[e, pl.ds(f * 32, 16)] = jnp.maximum(ev, 0.0)
                            mrow[e, pl.ds(f * 32 + 16, 16)] = jnp.maximum(
                                od, 0.0)
                    return 0
                lax.fori_loop(0, K // 2, ebody, 0)

                pltpu.sync_copy(mrow, acc.at[dstb.at[j]], add=True)
            return 0
        lax.fori_loop(0, CH // 2, cbody, 0)
        plsc.subcore_barrier()

        @pl.when(s < NS - 1)
        def _flush():
            pltpu.sync_copy(acc.at[pl.ds(s * STRIPE, STRIPE)],
                            part.at[c, fh, pl.ds(s * STRIPE, STRIPE)])

        @pl.when(s == NS - 1)
        def _flush_last():
            rem = N - (NS - 1) * STRIPE
            pltpu.sync_copy(acc.at[pl.ds((NS - 1) * STRIPE, rem)],
                            part.at[c, fh, pl.ds((NS - 1) * STRIPE, rem)])

        if fh == 0:
            plsc.subcore_barrier()


def _edge_aggregate(a0, a1, b0, b1, src2d, dst2d):
    mesh = plsc.VectorSubcoreMesh(core_axis_name="c", subcore_axis_name="s")
    fn = pl.kernel(
        _sc_body,
        out_type=jax.ShapeDtypeStruct((NC, 2, N, D), jnp.float32),
        mesh=mesh,
        compiler_params=pltpu.CompilerParams(use_tc_tiling_on_sc=False),
        scratch_types=[
            pltpu.VMEM_SHARED((NPAD, D), jnp.float32),
            pltpu.VMEM((CH, K), jnp.int32),
            pltpu.VMEM((CH, K), jnp.int32),
            pltpu.VMEM((2, K, D2), jnp.int32),
            pltpu.VMEM((2, K, D2), jnp.int32),
            pltpu.VMEM((K, D), jnp.float32),
            pltpu.VMEM((64, D), jnp.float32),
            pltpu.SemaphoreType.DMA,
            pltpu.SemaphoreType.DMA,
            pltpu.SemaphoreType.DMA,
            pltpu.SemaphoreType.DMA,
        ],
    )
    return fn(a0, a1, b0, b1, src2d, dst2d)


def _hid_body(p_ref, x_ref, w0_ref, w1_ref, w2_ref, b_ref, h_ref):
    m0 = p_ref[0, 0] + p_ref[1, 0]
    m1 = p_ref[0, 1] + p_ref[1, 1]
    dn = (((1,), (1,)), ((), ()))
    h = lax.dot_general(m0, w0_ref[...], dn, preferred_element_type=jnp.float32)
    h += lax.dot_general(m1, w1_ref[...], dn, preferred_element_type=jnp.float32)
    h += lax.dot_general(x_ref[...], w2_ref[...], dn,
                         preferred_element_type=jnp.float32)
    h_ref[...] = jnp.maximum(h + b_ref[...], 0.0)


def _update_nodes(part, x, wh0, wh1, wh2, b_hid):
    wspec = pl.BlockSpec((HID, D), lambda i: (0, 0))
    return pl.pallas_call(
        _hid_body,
        grid=(N // RB,),
        in_specs=[
            pl.BlockSpec((NC, 2, RB, D), lambda i: (0, 0, i, 0)),
            pl.BlockSpec((RB, D), lambda i: (i, 0)),
            wspec, wspec, wspec,
            pl.BlockSpec((1, HID), lambda i: (0, 0)),
        ],
        out_specs=pl.BlockSpec((RB, HID), lambda i: (i, 0)),
        out_shape=jax.ShapeDtypeStruct((N, HID), jnp.float32),
    )(part, x, wh0, wh1, wh2, b_hid)


@jax.jit
def _run(x, edge_index, W_msg, b_msg, W_hid, b_hid):
    a0, a1, b0, b1 = _project_ab(
        x, W_msg[:, :D], W_msg[:, D:], b_msg.reshape(1, MSG))
    src2d = edge_index[0].reshape(NC * NS, CH, K)
    dst2d = edge_index[1].reshape(NC * NS, CH, K)
    a0, a1, b0, b1 = (
        lax.bitcast_convert_type(t.reshape(N, D2, 2), jnp.int32)
        for t in (a0, a1, b0, b1))
    part = _edge_aggregate(a0, a1, b0, b1, src2d, dst2d)
    wm = W_hid[:, :MSG][:, UNPACK_PERM]
    return _update_nodes(part, x, wm[:, :D], wm[:, D:],
                         W_hid[:, MSG:], b_hid.reshape(1, HID))


def kernel(x, edge_index, node_ids, W_msg, b_msg, W_hid, b_hid):
    assert x.shape == (N, D) and edge_index.shape == (2, E)
    return _run(x, edge_index, W_msg, b_msg, W_hid, b_hid)


# speedup vs baseline: 1.8179x; 1.8179x over previous
import jax
import jax.numpy as jnp
from jax import lax
from jax.experimental import pallas as pl
from jax.experimental.pallas import tpu as pltpu
from jax.experimental.pallas import tpu_sc as plsc

N = 10000
E = 320000
D = 128
MSG = 256
HID = 384
NC = 2
NS = 16
K = 40
EPW = E // (NC * NS)
CH = EPW // K
G = 5
CHG = CH // G
NPAD = 10240
STRIPE = NPAD // NS
RB = 400


def _ab_body(x_ref, w1_ref, w2_ref, b_ref, a0_ref, a1_ref, b0_ref, b1_ref):
    xb = x_ref[...]
    dn = (((1,), (1,)), ((), ()))
    a = lax.dot_general(xb, w1_ref[...], dn, preferred_element_type=jnp.float32)
    b = lax.dot_general(xb, w2_ref[...], dn, preferred_element_type=jnp.float32)
    b = b + b_ref[...]
    a0_ref[...] = a[:, :D]
    a1_ref[...] = a[:, D:]
    b0_ref[...] = b[:, :D]
    b1_ref[...] = b[:, D:]


def _project_ab(x, w_src, w_dst, b_msg):
    half = pl.BlockSpec((RB, D), lambda i: (i, 0))
    return pl.pallas_call(
        _ab_body,
        grid=(N // RB,),
        in_specs=[
            pl.BlockSpec((RB, D), lambda i: (i, 0)),
            pl.BlockSpec((MSG, D), lambda i: (0, 0)),
            pl.BlockSpec((MSG, D), lambda i: (0, 0)),
            pl.BlockSpec((1, MSG), lambda i: (0, 0)),
        ],
        out_specs=[half, half, half, half],
        out_shape=[jax.ShapeDtypeStruct((N, D), jnp.float32)] * 4,
    )(x, w_src, w_dst, b_msg)


def _sc_body(a0, a1, b0, b1, src2d, dst2d, part,
             acc, srcb, dstb, arow, brow, mrow, zbuf,
             sem_a0, sem_a1, sem_b0, sem_b1, sem_m0, sem_m1):
    sems_a = (sem_a0, sem_a1)
    sems_b = (sem_b0, sem_b1)
    sems_m = (sem_m0, sem_m1)
    c = lax.axis_index("c")
    s = lax.axis_index("s")
    wid = c * NS + s

    zero = jnp.zeros((16,), jnp.float32)

    def zfill(i, _):
        for f in range(D // 16):
            zbuf[i, pl.ds(f * 16, 16)] = zero
        return 0
    lax.fori_loop(0, 32, zfill, 0)

    def fire(j, buf, a_hbm, b_hbm):
        pltpu.async_copy(a_hbm.at[srcb.at[j]], arow.at[buf], sems_a[buf])
        pltpu.async_copy(b_hbm.at[dstb.at[j]], brow.at[buf], sems_b[buf])

    def drain(j, buf, a_hbm, b_hbm):
        pltpu.make_async_copy(a_hbm.at[srcb.at[j]], arow.at[buf],
                              sems_a[buf]).wait()
        pltpu.make_async_copy(b_hbm.at[dstb.at[j]], brow.at[buf],
                              sems_b[buf]).wait()

    def retire_scatter(j, buf):
        pltpu.make_async_copy(mrow.at[buf], acc.at[dstb.at[j]],
                              sems_m[buf]).wait()

    for fh in range(2):
        a_hbm = a0 if fh == 0 else a1
        b_hbm = b0 if fh == 0 else b1

        def zbody(i, _):
            pltpu.sync_copy(zbuf, acc.at[pl.ds(s * STRIPE + i * 32, 32)])
            return 0
        lax.fori_loop(0, STRIPE // 32, zbody, 0)
        plsc.subcore_barrier()

        def gbody(g, _):
            @pl.when(g > 0)
            def _settle():
                retire_scatter(CHG - 2, 0)
                retire_scatter(CHG - 1, 1)

            pltpu.sync_copy(src2d.at[wid, g], srcb)
            pltpu.sync_copy(dst2d.at[wid, g], dstb)
            fire(0, 0, a_hbm, b_hbm)

            def cbody(j2, _):
                for b in range(2):
                    j = j2 * 2 + b
                    nxt = j + 1

                    @pl.when(nxt < CHG)
                    def _prefetch():
                        fire(nxt, 1 - b, a_hbm, b_hbm)

                    @pl.when(j >= 2)
                    def _retire():
                        retire_scatter(j - 2, b)

                    drain(j, b, a_hbm, b_hbm)

                    def ebody(e2, _):
                        for de in range(2):
                            e = e2 * 2 + de
                            for f in range(D // 16):
                                sl = pl.ds(f * 16, 16)
                                mrow[b, e, sl] = jnp.maximum(
                                    arow[b, e, sl] + brow[b, e, sl], 0.0)
                        return 0
                    lax.fori_loop(0, K // 2, ebody, 0)

                    pltpu.async_copy(mrow.at[b], acc.at[dstb.at[j]],
                                     sems_m[b], add=True)
                return 0
            lax.fori_loop(0, CHG // 2, cbody, 0)
            return 0
        lax.fori_loop(0, G, gbody, 0)
        retire_scatter(CHG - 2, 0)
        retire_scatter(CHG - 1, 1)
        plsc.subcore_barrier()

        @pl.when(s < NS - 1)
        def _flush():
            pltpu.sync_copy(acc.at[pl.ds(s * STRIPE, STRIPE)],
                            part.at[c, fh, pl.ds(s * STRIPE, STRIPE)])

        @pl.when(s == NS - 1)
        def _flush_last():
            rem = N - (NS - 1) * STRIPE
            pltpu.sync_copy(acc.at[pl.ds((NS - 1) * STRIPE, rem)],
                            part.at[c, fh, pl.ds((NS - 1) * STRIPE, rem)])

        if fh == 0:
            plsc.subcore_barrier()


def _edge_aggregate(a0, a1, b0, b1, src2d, dst2d):
    mesh = plsc.VectorSubcoreMesh(core_axis_name="c", subcore_axis_name="s")
    fn = pl.kernel(
        _sc_body,
        out_type=jax.ShapeDtypeStruct((NC, 2, N, D), jnp.float32),
        mesh=mesh,
        scratch_types=[
            pltpu.VMEM_SHARED((NPAD, D), jnp.float32),
            pltpu.VMEM((CHG, K), jnp.int32),
            pltpu.VMEM((CHG, K), jnp.int32),
            pltpu.VMEM((2, K, D), jnp.float32),
            pltpu.VMEM((2, K, D), jnp.float32),
            pltpu.VMEM((2, K, D), jnp.float32),
            pltpu.VMEM((32, D), jnp.float32),
            pltpu.SemaphoreType.DMA,
            pltpu.SemaphoreType.DMA,
            pltpu.SemaphoreType.DMA,
            pltpu.SemaphoreType.DMA,
            pltpu.SemaphoreType.DMA,
            pltpu.SemaphoreType.DMA,
        ],
    )
    return fn(a0, a1, b0, b1, src2d, dst2d)


def _hid_body(p_ref, x_ref, w0_ref, w1_ref, w2_ref, b_ref, h_ref):
    m0 = p_ref[0, 0] + p_ref[1, 0]
    m1 = p_ref[0, 1] + p_ref[1, 1]
    dn = (((1,), (1,)), ((), ()))
    h = lax.dot_general(m0, w0_ref[...], dn, preferred_element_type=jnp.float32)
    h += lax.dot_general(m1, w1_ref[...], dn, preferred_element_type=jnp.float32)
    h += lax.dot_general(x_ref[...], w2_ref[...], dn,
                         preferred_element_type=jnp.float32)
    h_ref[...] = jnp.maximum(h + b_ref[...], 0.0)


def _update_nodes(part, x, wh0, wh1, wh2, b_hid):
    wspec = pl.BlockSpec((HID, D), lambda i: (0, 0))
    return pl.pallas_call(
        _hid_body,
        grid=(N // RB,),
        in_specs=[
            pl.BlockSpec((NC, 2, RB, D), lambda i: (0, 0, i, 0)),
            pl.BlockSpec((RB, D), lambda i: (i, 0)),
            wspec, wspec, wspec,
            pl.BlockSpec((1, HID), lambda i: (0, 0)),
        ],
        out_specs=pl.BlockSpec((RB, HID), lambda i: (i, 0)),
        out_shape=jax.ShapeDtypeStruct((N, HID), jnp.float32),
    )(part, x, wh0, wh1, wh2, b_hid)


@jax.jit
def _run(x, edge_index, W_msg, b_msg, W_hid, b_hid):
    a0, a1, b0, b1 = _project_ab(
        x, W_msg[:, :D], W_msg[:, D:], b_msg.reshape(1, MSG))
    src2d = edge_index[0].reshape(NC * NS, G, CHG, K)
    dst2d = edge_index[1].reshape(NC * NS, G, CHG, K)
    part = _edge_aggregate(a0, a1, b0, b1, src2d, dst2d)
    return _update_nodes(part, x, W_hid[:, :D], W_hid[:, D:2 * D],
                         W_hid[:, 2 * D:], b_hid.reshape(1, HID))


def kernel(x, edge_index, node_ids, W_msg, b_msg, W_hid, b_hid):
    assert x.shape == (N, D) and edge_index.shape == (2, E)
    return _run(x, edge_index, W_msg, b_msg, W_hid, b_hid)


# speedup vs baseline: 1.8252x; 1.0040x over previous
import jax
import jax.numpy as jnp
from jax import lax
from jax.experimental import pallas as pl
from jax.experimental.pallas import tpu as pltpu
from jax.experimental.pallas import tpu_sc as plsc

N = 10000
E = 320000
D = 128
MSG = 256
HID = 384
NC = 2
NS = 16
K = 40
EPW = E // (NC * NS)
CH = EPW // K
G = 5
CHG = CH // G
NPAD = 10240
STRIPE = NPAD // NS
RB = 400


def _ab_body(x_ref, w1_ref, w2_ref, b_ref, a0_ref, a1_ref, b0_ref, b1_ref):
    xb = x_ref[...]
    dn = (((1,), (1,)), ((), ()))
    a = lax.dot_general(xb, w1_ref[...], dn, preferred_element_type=jnp.float32)
    b = lax.dot_general(xb, w2_ref[...], dn, preferred_element_type=jnp.float32)
    b = b + b_ref[...]
    a0_ref[...] = a[:, :D]
    a1_ref[...] = a[:, D:]
    b0_ref[...] = b[:, :D]
    b1_ref[...] = b[:, D:]


def _project_ab(x, w_src, w_dst, b_msg):
    half = pl.BlockSpec((RB, D), lambda i: (i, 0))
    return pl.pallas_call(
        _ab_body,
        grid=(N // RB,),
        in_specs=[
            pl.BlockSpec((RB, D), lambda i: (i, 0)),
            pl.BlockSpec((MSG, D), lambda i: (0, 0)),
            pl.BlockSpec((MSG, D), lambda i: (0, 0)),
            pl.BlockSpec((1, MSG), lambda i: (0, 0)),
        ],
        out_specs=[half, half, half, half],
        out_shape=[jax.ShapeDtypeStruct((N, D), jnp.float32)] * 4,
    )(x, w_src, w_dst, b_msg)


def _sc_body(a0, a1, b0, b1, src2d, dst2d, part,
             acc, srcb, dstb, arow, brow, mrow, zbuf,
             sem_a0, sem_a1, sem_b0, sem_b1, sem_m0, sem_m1):
    sems_a = (sem_a0, sem_a1)
    sems_b = (sem_b0, sem_b1)
    sems_m = (sem_m0, sem_m1)
    c = lax.axis_index("c")
    s = lax.axis_index("s")
    wid = c * NS + s

    zero = jnp.zeros((16,), jnp.float32)

    def zfill(i, _):
        for f in range(D // 16):
            zbuf[i, pl.ds(f * 16, 16)] = zero
        return 0
    lax.fori_loop(0, 32, zfill, 0)

    def fire(j, buf, a_hbm, b_hbm):
        pltpu.async_copy(a_hbm.at[srcb.at[j]], arow.at[buf], sems_a[buf])
        pltpu.async_copy(b_hbm.at[dstb.at[j]], brow.at[buf], sems_b[buf])

    def drain(j, buf, a_hbm, b_hbm):
        pltpu.make_async_copy(a_hbm.at[srcb.at[j]], arow.at[buf],
                              sems_a[buf]).wait()
        pltpu.make_async_copy(b_hbm.at[dstb.at[j]], brow.at[buf],
                              sems_b[buf]).wait()

    def retire_scatter(j, buf):
        pltpu.make_async_copy(mrow.at[buf], acc.at[dstb.at[j]],
                              sems_m[buf]).wait()

    for fh in range(2):
        a_hbm = a0 if fh == 0 else a1
        b_hbm = b0 if fh == 0 else b1

        nz = STRIPE // 32

        def zbody(i, _):
            pltpu.async_copy(zbuf, acc.at[pl.ds(s * STRIPE + i * 32, 32)],
                             sem_m0)

            @pl.when(i > 0)
            def _zwait():
                pltpu.make_async_copy(
                    zbuf, acc.at[pl.ds(s * STRIPE, 32)], sem_m0).wait()
            return 0
        lax.fori_loop(0, nz, zbody, 0)
        pltpu.make_async_copy(zbuf, acc.at[pl.ds(s * STRIPE, 32)],
                              sem_m0).wait()
        plsc.subcore_barrier()

        def gbody(g, _):
            @pl.when(g > 0)
            def _settle():
                retire_scatter(CHG - 2, 0)
                retire_scatter(CHG - 1, 1)

            pltpu.sync_copy(src2d.at[wid, g], srcb)
            pltpu.sync_copy(dst2d.at[wid, g], dstb)
            fire(0, 0, a_hbm, b_hbm)

            def cbody(j2, _):
                for b in range(2):
                    j = j2 * 2 + b
                    nxt = j + 1

                    @pl.when(nxt < CHG)
                    def _prefetch():
                        fire(nxt, 1 - b, a_hbm, b_hbm)

                    @pl.when(j >= 2)
                    def _retire():
                        retire_scatter(j - 2, b)

                    drain(j, b, a_hbm, b_hbm)

                    def ebody(e4, _):
                        for de in range(4):
                            e = e4 * 4 + de
                            for f in range(D // 16):
                                sl = pl.ds(f * 16, 16)
                                mrow[b, e, sl] = jnp.maximum(
                                    arow[b, e, sl] + brow[b, e, sl], 0.0)
                        return 0
                    lax.fori_loop(0, K // 4, ebody, 0)

                    pltpu.async_copy(mrow.at[b], acc.at[dstb.at[j]],
                                     sems_m[b], add=True)
                return 0
            lax.fori_loop(0, CHG // 2, cbody, 0)
            return 0
        lax.fori_loop(0, G, gbody, 0)
        retire_scatter(CHG - 2, 0)
        retire_scatter(CHG - 1, 1)
        plsc.subcore_barrier()

        @pl.when(s < NS - 1)
        def _flush():
            pltpu.sync_copy(acc.at[pl.ds(s * STRIPE, STRIPE)],
                            part.at[c, fh, pl.ds(s * STRIPE, STRIPE)])

        @pl.when(s == NS - 1)
        def _flush_last():
            rem = N - (NS - 1) * STRIPE
            pltpu.sync_copy(acc.at[pl.ds((NS - 1) * STRIPE, rem)],
                            part.at[c, fh, pl.ds((NS - 1) * STRIPE, rem)])

        if fh == 0:
            plsc.subcore_barrier()


def _edge_aggregate(a0, a1, b0, b1, src2d, dst2d):
    mesh = plsc.VectorSubcoreMesh(core_axis_name="c", subcore_axis_name="s")
    fn = pl.kernel(
        _sc_body,
        out_type=jax.ShapeDtypeStruct((NC, 2, N, D), jnp.float32),
        mesh=mesh,
        scratch_types=[
            pltpu.VMEM_SHARED((NPAD, D), jnp.float32),
            pltpu.VMEM((CHG, K), jnp.int32),
            pltpu.VMEM((CHG, K), jnp.int32),
            pltpu.VMEM((2, K, D), jnp.float32),
            pltpu.VMEM((2, K, D), jnp.float32),
            pltpu.VMEM((2, K, D), jnp.float32),
            pltpu.VMEM((32, D), jnp.float32),
            pltpu.SemaphoreType.DMA,
            pltpu.SemaphoreType.DMA,
            pltpu.SemaphoreType.DMA,
            pltpu.SemaphoreType.DMA,
            pltpu.SemaphoreType.DMA,
            pltpu.SemaphoreType.DMA,
        ],
    )
    return fn(a0, a1, b0, b1, src2d, dst2d)


def _hid_body(p_ref, x_ref, w0_ref, w1_ref, w2_ref, b_ref, h_ref):
    m0 = p_ref[0, 0] + p_ref[1, 0]
    m1 = p_ref[0, 1] + p_ref[1, 1]
    dn = (((1,), (1,)), ((), ()))
    h = lax.dot_general(m0, w0_ref[...], dn, preferred_element_type=jnp.float32)
    h += lax.dot_general(m1, w1_ref[...], dn, preferred_element_type=jnp.float32)
    h += lax.dot_general(x_ref[...], w2_ref[...], dn,
                         preferred_element_type=jnp.float32)
    h_ref[...] = jnp.maximum(h + b_ref[...], 0.0)


def _update_nodes(part, x, wh0, wh1, wh2, b_hid):
    wspec = pl.BlockSpec((HID, D), lambda i: (0, 0))
    return pl.pallas_call(
        _hid_body,
        grid=(N // RB,),
        in_specs=[
            pl.BlockSpec((NC, 2, RB, D), lambda i: (0, 0, i, 0)),
            pl.BlockSpec((RB, D), lambda i: (i, 0)),
            wspec, wspec, wspec,
            pl.BlockSpec((1, HID), lambda i: (0, 0)),
        ],
        out_specs=pl.BlockSpec((RB, HID), lambda i: (i, 0)),
        out_shape=jax.ShapeDtypeStruct((N, HID), jnp.float32),
    )(part, x, wh0, wh1, wh2, b_hid)


@jax.jit
def _run(x, edge_index, W_msg, b_msg, W_hid, b_hid):
    a0, a1, b0, b1 = _project_ab(
        x, W_msg[:, :D], W_msg[:, D:], b_msg.reshape(1, MSG))
    src2d = edge_index[0].reshape(NC * NS, G, CHG, K)
    dst2d = edge_index[1].reshape(NC * NS, G, CHG, K)
    part = _edge_aggregate(a0, a1, b0, b1, src2d, dst2d)
    return _update_nodes(part, x, W_hid[:, :D], W_hid[:, D:2 * D],
                         W_hid[:, 2 * D:], b_hid.reshape(1, HID))


def kernel(x, edge_index, node_ids, W_msg, b_msg, W_hid, b_hid):
    assert x.shape == (N, D) and edge_index.shape == (2, E)
    return _run(x, edge_index, W_msg, b_msg, W_hid, b_hid)


# speedup vs baseline: 1.8766x; 1.0282x over previous
import jax
import jax.numpy as jnp
from jax import lax
from jax.experimental import pallas as pl
from jax.experimental.pallas import tpu as pltpu
from jax.experimental.pallas import tpu_sc as plsc

N = 10000
E = 320000
D = 128
MSG = 256
HID = 384
NC = 2
NS = 16
K = 40
EPW = E // NS
CH = EPW // K
G = 10
CHG = CH // G
NPAD = 10240
STRIPE = NPAD // NS
RB = 400


def _ab_body(x_ref, w1_ref, w2_ref, b_ref, a0_ref, a1_ref, b0_ref, b1_ref):
    xb = x_ref[...]
    dn = (((1,), (1,)), ((), ()))
    a = lax.dot_general(xb, w1_ref[...], dn, preferred_element_type=jnp.float32)
    b = lax.dot_general(xb, w2_ref[...], dn, preferred_element_type=jnp.float32)
    b = b + b_ref[...]
    a0_ref[...] = a[:, :D]
    a1_ref[...] = a[:, D:]
    b0_ref[...] = b[:, :D]
    b1_ref[...] = b[:, D:]


def _project_ab(x, w_src, w_dst, b_msg):
    half = pl.BlockSpec((RB, D), lambda i: (i, 0))
    return pl.pallas_call(
        _ab_body,
        grid=(N // RB,),
        in_specs=[
            pl.BlockSpec((RB, D), lambda i: (i, 0)),
            pl.BlockSpec((MSG, D), lambda i: (0, 0)),
            pl.BlockSpec((MSG, D), lambda i: (0, 0)),
            pl.BlockSpec((1, MSG), lambda i: (0, 0)),
        ],
        out_specs=[half, half, half, half],
        out_shape=[jax.ShapeDtypeStruct((N, D), jnp.float32)] * 4,
    )(x, w_src, w_dst, b_msg)


def _sc_body(a0, a1, b0, b1, src2d, dst2d, part,
             acc, srcb, dstb, arow, brow, mrow, zbuf,
             sem_a0, sem_a1, sem_b0, sem_b1, sem_m0, sem_m1):
    sems_a = (sem_a0, sem_a1)
    sems_b = (sem_b0, sem_b1)
    sems_m = (sem_m0, sem_m1)
    c = lax.axis_index("c")
    s = lax.axis_index("s")

    zero = jnp.zeros((16,), jnp.float32)

    def zfill(i, _):
        for f in range(D // 16):
            zbuf[i, pl.ds(f * 16, 16)] = zero
        return 0
    lax.fori_loop(0, 32, zfill, 0)

    def fire(j, buf, a_hbm, b_hbm):
        pltpu.async_copy(a_hbm.at[srcb.at[j]], arow.at[buf], sems_a[buf])
        pltpu.async_copy(b_hbm.at[dstb.at[j]], brow.at[buf], sems_b[buf])

    def drain(j, buf, a_hbm, b_hbm):
        pltpu.make_async_copy(a_hbm.at[srcb.at[j]], arow.at[buf],
                              sems_a[buf]).wait()
        pltpu.make_async_copy(b_hbm.at[dstb.at[j]], brow.at[buf],
                              sems_b[buf]).wait()

    def retire_scatter(j, buf):
        pltpu.make_async_copy(mrow.at[buf], acc.at[dstb.at[j]],
                              sems_m[buf]).wait()

    def _pass(a_hbm, b_hbm):
        nz = STRIPE // 32

        def zbody(i, _):
            pltpu.async_copy(zbuf, acc.at[pl.ds(s * STRIPE + i * 32, 32)],
                             sem_m0)

            @pl.when(i > 0)
            def _zwait():
                pltpu.make_async_copy(
                    zbuf, acc.at[pl.ds(s * STRIPE, 32)], sem_m0).wait()
            return 0
        lax.fori_loop(0, nz, zbody, 0)
        pltpu.make_async_copy(zbuf, acc.at[pl.ds(s * STRIPE, 32)],
                              sem_m0).wait()
        plsc.subcore_barrier()

        def gbody(g, _):
            @pl.when(g > 0)
            def _settle():
                retire_scatter(CHG - 2, 0)
                retire_scatter(CHG - 1, 1)

            pltpu.sync_copy(src2d.at[s, g], srcb)
            pltpu.sync_copy(dst2d.at[s, g], dstb)
            fire(0, 0, a_hbm, b_hbm)

            def cbody(j2, _):
                for b in range(2):
                    j = j2 * 2 + b
                    nxt = j + 1

                    @pl.when(nxt < CHG)
                    def _prefetch():
                        fire(nxt, 1 - b, a_hbm, b_hbm)

                    @pl.when(j >= 2)
                    def _retire():
                        retire_scatter(j - 2, b)

                    drain(j, b, a_hbm, b_hbm)

                    def ebody(e4, _):
                        for de in range(4):
                            e = e4 * 4 + de
                            for f in range(D // 16):
                                sl = pl.ds(f * 16, 16)
                                mrow[b, e, sl] = jnp.maximum(
                                    arow[b, e, sl] + brow[b, e, sl], 0.0)
                        return 0
                    lax.fori_loop(0, K // 4, ebody, 0)

                    pltpu.async_copy(mrow.at[b], acc.at[dstb.at[j]],
                                     sems_m[b], add=True)
                return 0
            lax.fori_loop(0, CHG // 2, cbody, 0)
            return 0
        lax.fori_loop(0, G, gbody, 0)
        retire_scatter(CHG - 2, 0)
        retire_scatter(CHG - 1, 1)
        plsc.subcore_barrier()

        @pl.when(s < NS - 1)
        def _flush():
            pltpu.sync_copy(acc.at[pl.ds(s * STRIPE, STRIPE)],
                            part.at[c, pl.ds(s * STRIPE, STRIPE)])

        @pl.when(s == NS - 1)
        def _flush_last():
            rem = N - (NS - 1) * STRIPE
            pltpu.sync_copy(acc.at[pl.ds((NS - 1) * STRIPE, rem)],
                            part.at[c, pl.ds((NS - 1) * STRIPE, rem)])

    for ch in range(2):
        @pl.when(c == ch)
        def _run_pass(ch=ch):
            _pass((a0, a1)[ch], (b0, b1)[ch])


def _edge_aggregate(a0, a1, b0, b1, src2d, dst2d):
    mesh = plsc.VectorSubcoreMesh(core_axis_name="c", subcore_axis_name="s")
    fn = pl.kernel(
        _sc_body,
        out_type=jax.ShapeDtypeStruct((NC, N, D), jnp.float32),
        mesh=mesh,
        scratch_types=[
            pltpu.VMEM_SHARED((NPAD, D), jnp.float32),
            pltpu.VMEM((CHG, K), jnp.int32),
            pltpu.VMEM((CHG, K), jnp.int32),
            pltpu.VMEM((2, K, D), jnp.float32),
            pltpu.VMEM((2, K, D), jnp.float32),
            pltpu.VMEM((2, K, D), jnp.float32),
            pltpu.VMEM((32, D), jnp.float32),
            pltpu.SemaphoreType.DMA,
            pltpu.SemaphoreType.DMA,
            pltpu.SemaphoreType.DMA,
            pltpu.SemaphoreType.DMA,
            pltpu.SemaphoreType.DMA,
            pltpu.SemaphoreType.DMA,
        ],
    )
    return fn(a0, a1, b0, b1, src2d, dst2d)


def _hid_body(p_ref, x_ref, w0_ref, w1_ref, w2_ref, b_ref, h_ref):
    m0 = p_ref[0]
    m1 = p_ref[1]
    dn = (((1,), (1,)), ((), ()))
    h = lax.dot_general(m0, w0_ref[...], dn, preferred_element_type=jnp.float32)
    h += lax.dot_general(m1, w1_ref[...], dn, preferred_element_type=jnp.float32)
    h += lax.dot_general(x_ref[...], w2_ref[...], dn,
                         preferred_element_type=jnp.float32)
    h_ref[...] = jnp.maximum(h + b_ref[...], 0.0)


def _update_nodes(part, x, wh0, wh1, wh2, b_hid):
    wspec = pl.BlockSpec((HID, D), lambda i: (0, 0))
    return pl.pallas_call(
        _hid_body,
        grid=(N // RB,),
        in_specs=[
            pl.BlockSpec((NC, RB, D), lambda i: (0, i, 0)),
            pl.BlockSpec((RB, D), lambda i: (i, 0)),
            wspec, wspec, wspec,
            pl.BlockSpec((1, HID), lambda i: (0, 0)),
        ],
        out_specs=pl.BlockSpec((RB, HID), lambda i: (i, 0)),
        out_shape=jax.ShapeDtypeStruct((N, HID), jnp.float32),
    )(part, x, wh0, wh1, wh2, b_hid)


@jax.jit
def _run(x, edge_index, W_msg, b_msg, W_hid, b_hid):
    a0, a1, b0, b1 = _project_ab(
        x, W_msg[:, :D], W_msg[:, D:], b_msg.reshape(1, MSG))
    src2d = edge_index[0].reshape(NS, G, CHG, K)
    dst2d = edge_index[1].reshape(NS, G, CHG, K)
    part = _edge_aggregate(a0, a1, b0, b1, src2d, dst2d)
    return _update_nodes(part, x, W_hid[:, :D], W_hid[:, D:2 * D],
                         W_hid[:, 2 * D:], b_hid.reshape(1, HID))


def kernel(x, edge_index, node_ids, W_msg, b_msg, W_hid, b_hid):
    assert x.shape == (N, D) and edge_index.shape == (2, E)
    return _run(x, edge_index, W_msg, b_msg, W_hid, b_hid)
